# baseline (speedup 1.0000x reference)
import jax
import jax.numpy as jnp
from jax.experimental import pallas as pl
from jax.experimental.pallas import tpu as pltpu

_EPS_FUSED = 1e-4


def _mm(x, w):
    h, g, k = x.shape
    y = jnp.dot(x.reshape(h * g, k), w, preferred_element_type=jnp.float32)
    return y.reshape(h, g, 128)


def _bifpn_body(wn_ref, x0_ref, x1_ref, x2_ref, lw0_ref, lw1_ref, lw2_ref,
                lb_ref, rs_ref, pws_ref, dws_ref, bns_ref, bnt_ref,
                o0_ref, o1_ref, o2_ref):
    re_m = rs_ref[0:128, :]
    ro_m = rs_ref[128:256, :]
    da_m = rs_ref[256:384, :]
    db_m = rs_ref[384:512, :]

    def up2(x):
        xx = jnp.concatenate([x, x], axis=0)
        return jnp.concatenate([_mm(xx, re_m), _mm(xx, ro_m)], axis=1)

    def down2(x):
        h, g, _ = x.shape
        a = x[:h // 2, :g // 2]
        b = x[:h // 2, g // 2:]
        return _mm(a, da_m) + _mm(b, db_m)

    def fused(plist, k):
        acc = wn_ref[k, 0] * plist[0]
        for j in range(1, len(plist)):
            acc = acc + wn_ref[k, j] * plist[j]
        x = acc * dws_ref[k:k + 1, :]
        y = _mm(x, pws_ref[128 * k:128 * (k + 1), :])
        y = y * bns_ref[k:k + 1, :] + bnt_ref[k:k + 1, :]
        return jnp.maximum(y, 0.0)

    x0 = x0_ref[0]
    x1 = x1_ref[0]
    p0 = jnp.concatenate(
        [_mm(x0, lw0_ref[0:128, :]), _mm(x0, lw0_ref[128:256, :]),
         _mm(x0, lw0_ref[256:384, :]), _mm(x0, lw0_ref[384:512, :])],
        axis=1) + lb_ref[0:1, :]
    p1 = jnp.concatenate(
        [_mm(x1, lw1_ref[0:128, :]), _mm(x1, lw1_ref[128:256, :])],
        axis=1) + lb_ref[1:2, :]
    p2 = _mm(x2_ref[0], lw2_ref[...]) + lb_ref[2:3, :]

    ps = [p2, p1, p0]
    for l in range(2):
        base = 4 * l
        a2 = ps[0]
        a1 = fused([ps[1], up2(a2)], base + 0)
        a0 = fused([ps[2], up2(a1)], base + 1)
        o1 = fused([ps[1], a1, down2(a0)], base + 2)
        o2 = fused([ps[0], a2, down2(o1)], base + 3)
        ps = [o2, o1, a0]

    o2_ref[0] = ps[0]
    o1_ref[0] = ps[1]
    o0_ref[0] = ps[2]


def _kron4(m):
    return jnp.kron(jnp.eye(4, dtype=m.dtype), m)


def _tile4(v):
    return jnp.tile(v, 4)


def _wn(w_raw):
    w = jnp.maximum(w_raw, 0.0)
    w = w / (jnp.sum(w) + _EPS_FUSED)
    return jnp.pad(w, (0, 3 - w.shape[0]))


def kernel(input0, input1, input2, plat0_w, plat0_b, plat1_w, plat1_b,
           plat2_w, plat2_b, L0_lat1_0_dw, L0_lat1_0_pw, L0_lat1_0_bn_scale,
           L0_lat1_0_bn_shift, L0_lat1_1_dw, L0_lat1_1_pw, L0_lat1_1_bn_scale,
           L0_lat1_1_bn_shift, L0_lat2_0_dw, L0_lat2_0_pw, L0_lat2_0_bn_scale,
           L0_lat2_0_bn_shift, L0_lat2_1_dw, L0_lat2_1_pw, L0_lat2_1_bn_scale,
           L0_lat2_1_bn_shift, L0_wtd_0, L0_wtd_1, L0_wbu_0, L0_wbu_1,
           L1_lat1_0_dw, L1_lat1_0_pw, L1_lat1_0_bn_scale, L1_lat1_0_bn_shift,
           L1_lat1_1_dw, L1_lat1_1_pw, L1_lat1_1_bn_scale, L1_lat1_1_bn_shift,
           L1_lat2_0_dw, L1_lat2_0_pw, L1_lat2_0_bn_scale, L1_lat2_0_bn_shift,
           L1_lat2_1_dw, L1_lat2_1_pw, L1_lat2_1_bn_scale, L1_lat2_1_bn_shift,
           L1_wtd_0, L1_wtd_1, L1_wbu_0, L1_wbu_1):
    n = input0.shape[0]
    f32 = jnp.float32

    xr = input0.reshape(n, 8, 32, 2, 2, 8, 16)
    x0 = jnp.transpose(xr, (0, 4, 3, 2, 5, 6, 1)).reshape(n, 128, 8, 128)
    xr = input1.reshape(n, 16, 32, 2, 8, 8)
    x1 = jnp.transpose(xr, (0, 3, 2, 4, 5, 1)).reshape(n, 64, 8, 128)
    xp = jnp.pad(input2, ((0, 0), (0, 8), (0, 0), (0, 0)))
    x2 = jnp.transpose(xp, (0, 2, 3, 1)).reshape(n, 32, 8, 128)

    eye16 = jnp.eye(16, dtype=f32)
    eye8 = jnp.eye(8, dtype=f32)
    lw0 = jnp.concatenate(
        [jnp.kron(eye16[:, 4 * r:4 * r + 4], plat0_w) for r in (0, 2, 1, 3)],
        axis=0)
    lw1 = jnp.concatenate(
        [jnp.kron(eye8[:, 4 * r:4 * r + 4], plat1_w) for r in (0, 1)],
        axis=0)
    lw2 = _kron4(jnp.pad(plat2_w, ((0, 8), (0, 0))))
    lb = jnp.stack([_tile4(plat0_b), _tile4(plat1_b), _tile4(plat2_b)])

    se = jnp.array([[1, 1, 0, 0], [0, 0, 1, 1], [0, 0, 0, 0], [0, 0, 0, 0]], f32)
    so = jnp.array([[0, 0, 0, 0], [0, 0, 0, 0], [1, 1, 0, 0], [0, 0, 1, 1]], f32)
    sa = jnp.array([[1, 0, 0, 0], [0, 0, 0, 0], [0, 1, 0, 0], [0, 0, 0, 0]], f32)
    sb = jnp.array([[0, 0, 1, 0], [0, 0, 0, 0], [0, 0, 0, 1], [0, 0, 0, 0]], f32)
    eye32 = jnp.eye(32, dtype=f32)
    rs = jnp.concatenate([jnp.kron(s, eye32) for s in (se, so, sa, sb)], axis=0)

    steps = [
        (L0_lat1_0_dw, L0_lat1_0_pw, L0_lat1_0_bn_scale, L0_lat1_0_bn_shift, L0_wtd_0),
        (L0_lat1_1_dw, L0_lat1_1_pw, L0_lat1_1_bn_scale, L0_lat1_1_bn_shift, L0_wtd_1),
        (L0_lat2_0_dw, L0_lat2_0_pw, L0_lat2_0_bn_scale, L0_lat2_0_bn_shift, L0_wbu_0),
        (L0_lat2_1_dw, L0_lat2_1_pw, L0_lat2_1_bn_scale, L0_lat2_1_bn_shift, L0_wbu_1),
        (L1_lat1_0_dw, L1_lat1_0_pw, L1_lat1_0_bn_scale, L1_lat1_0_bn_shift, L1_wtd_0),
        (L1_lat1_1_dw, L1_lat1_1_pw, L1_lat1_1_bn_scale, L1_lat1_1_bn_shift, L1_wtd_1),
        (L1_lat2_0_dw, L1_lat2_0_pw, L1_lat2_0_bn_scale, L1_lat2_0_bn_shift, L1_wbu_0),
        (L1_lat2_1_dw, L1_lat2_1_pw, L1_lat2_1_bn_scale, L1_lat2_1_bn_shift, L1_wbu_1),
    ]
    pws = jnp.concatenate([_kron4(s[1]) for s in steps], axis=0)
    dws = jnp.stack([_tile4(s[0]) for s in steps])
    bns = jnp.stack([_tile4(s[2]) for s in steps])
    bnt = jnp.stack([_tile4(s[3]) for s in steps])
    wn = jnp.stack([_wn(s[4]) for s in steps])

    const = lambda i, w_: (0, 0)
    o0, o1, o2 = pl.pallas_call(
        _bifpn_body,
        out_shape=[
            jax.ShapeDtypeStruct((n, 128, 32, 128), f32),
            jax.ShapeDtypeStruct((n, 64, 16, 128), f32),
            jax.ShapeDtypeStruct((n, 32, 8, 128), f32),
        ],
        grid_spec=pltpu.PrefetchScalarGridSpec(
            num_scalar_prefetch=1,
            grid=(n,),
            in_specs=[
                pl.BlockSpec((1, 128, 8, 128), lambda i, w_: (i, 0, 0, 0)),
                pl.BlockSpec((1, 64, 8, 128), lambda i, w_: (i, 0, 0, 0)),
                pl.BlockSpec((1, 32, 8, 128), lambda i, w_: (i, 0, 0, 0)),
                pl.BlockSpec((512, 128), const),
                pl.BlockSpec((256, 128), const),
                pl.BlockSpec((128, 128), const),
                pl.BlockSpec((3, 128), const),
                pl.BlockSpec((512, 128), const),
                pl.BlockSpec((1024, 128), const),
                pl.BlockSpec((8, 128), const),
                pl.BlockSpec((8, 128), const),
                pl.BlockSpec((8, 128), const),
            ],
            out_specs=[
                pl.BlockSpec((1, 128, 32, 128), lambda i, w_: (i, 0, 0, 0)),
                pl.BlockSpec((1, 64, 16, 128), lambda i, w_: (i, 0, 0, 0)),
                pl.BlockSpec((1, 32, 8, 128), lambda i, w_: (i, 0, 0, 0)),
            ],
        ),
        compiler_params=pltpu.CompilerParams(
            dimension_semantics=("parallel",),
            vmem_limit_bytes=64 * 1024 * 1024,
        ),
    )(wn, x0, x1, x2, lw0, lw1, lw2, lb, rs, pws, dws, bns, bnt)

    y0 = o0.reshape(n, 2, 2, 32, 2, 2, 8, 4, 32)
    y0 = jnp.transpose(y0, (0, 8, 3, 2, 1, 6, 5, 4, 7)).reshape(n, 32, 128, 128)
    y1 = o1.reshape(n, 2, 32, 2, 8, 4, 32)
    y1 = jnp.transpose(y1, (0, 6, 2, 1, 4, 3, 5)).reshape(n, 32, 64, 64)
    y2 = o2.reshape(n, 32, 8, 4, 32)
    y2 = jnp.transpose(y2, (0, 4, 1, 2, 3)).reshape(n, 32, 32, 32)
    return [y2, y1, y0]


# speedup vs baseline: 1.1766x; 1.1766x over previous
import jax
import jax.numpy as jnp
from jax.experimental import pallas as pl
from jax.experimental.pallas import tpu as pltpu

_EPS_FUSED = 1e-4


def _mm(x, w):
    h, g, k = x.shape
    y = jnp.dot(x.reshape(h * g, k), w, preferred_element_type=jnp.float32)
    return y.reshape(h, g, 128)


def _bifpn_body(wn_ref, x0_ref, x1_ref, x2_ref, lw0_ref, lw1_ref, lw2_ref,
                lb_ref, rs_ref, pws_ref, dws_ref, bns_ref, bnt_ref,
                o0_ref, o1_ref, o2_ref):
    re_m = rs_ref[0:128, :]
    ro_m = rs_ref[128:256, :]
    da_m = rs_ref[256:384, :]
    db_m = rs_ref[384:512, :]

    def up2(x):
        xx = jnp.concatenate([x, x], axis=0)
        return jnp.concatenate([_mm(xx, re_m), _mm(xx, ro_m)], axis=1)

    def down2(x):
        h, g, _ = x.shape
        a = x[:h // 2, :g // 2]
        b = x[:h // 2, g // 2:]
        return _mm(a, da_m) + _mm(b, db_m)

    def fused(plist, k):
        acc = wn_ref[k, 0] * plist[0]
        for j in range(1, len(plist)):
            acc = acc + wn_ref[k, j] * plist[j]
        x = acc * dws_ref[k:k + 1, :]
        y = _mm(x, pws_ref[128 * k:128 * (k + 1), :])
        y = y * bns_ref[k:k + 1, :] + bnt_ref[k:k + 1, :]
        return jnp.maximum(y, 0.0)

    x0r = x0_ref[0]
    x0s = jnp.concatenate(
        [x0r[:, 0], x0r[:, 2], x0r[:, 1], x0r[:, 3]], axis=0)
    x1r = x1_ref[0]
    x1s = jnp.concatenate([x1r[:, 0], x1r[:, 1]], axis=0)

    p0 = jnp.concatenate(
        [_mm(x0s, lw0_ref[0:128, :]), _mm(x0s, lw0_ref[128:256, :]),
         _mm(x0s, lw0_ref[256:384, :]), _mm(x0s, lw0_ref[384:512, :])],
        axis=1) + lb_ref[0:1, :]
    p1 = jnp.concatenate(
        [_mm(x1s, lw1_ref[0:128, :]), _mm(x1s, lw1_ref[128:256, :])],
        axis=1) + lb_ref[1:2, :]
    p2 = _mm(x2_ref[0], lw2_ref[...]) + lb_ref[2:3, :]

    ps = [p2, p1, p0]
    for l in range(2):
        base = 4 * l
        a2 = ps[0]
        a1 = fused([ps[1], up2(a2)], base + 0)
        a0 = fused([ps[2], up2(a1)], base + 1)
        o1 = fused([ps[1], a1, down2(a0)], base + 2)
        o2 = fused([ps[0], a2, down2(o1)], base + 3)
        ps = [o2, o1, a0]

    s0, s1 = ps[2], ps[1]
    o0_ref[0, :, 0] = s0[0:32]
    o0_ref[0, :, 2] = s0[32:64]
    o0_ref[0, :, 1] = s0[64:96]
    o0_ref[0, :, 3] = s0[96:128]
    o1_ref[0, :, 0] = s1[0:32]
    o1_ref[0, :, 1] = s1[32:64]
    o2_ref[0] = ps[0]


def _kron4(m):
    return jnp.kron(jnp.eye(4, dtype=m.dtype), m)


def _tile4(v):
    return jnp.tile(v, 4)


def _wn(w_raw):
    w = jnp.maximum(w_raw, 0.0)
    w = w / (jnp.sum(w) + _EPS_FUSED)
    return jnp.pad(w, (0, 3 - w.shape[0]))


def kernel(input0, input1, input2, plat0_w, plat0_b, plat1_w, plat1_b,
           plat2_w, plat2_b, L0_lat1_0_dw, L0_lat1_0_pw, L0_lat1_0_bn_scale,
           L0_lat1_0_bn_shift, L0_lat1_1_dw, L0_lat1_1_pw, L0_lat1_1_bn_scale,
           L0_lat1_1_bn_shift, L0_lat2_0_dw, L0_lat2_0_pw, L0_lat2_0_bn_scale,
           L0_lat2_0_bn_shift, L0_lat2_1_dw, L0_lat2_1_pw, L0_lat2_1_bn_scale,
           L0_lat2_1_bn_shift, L0_wtd_0, L0_wtd_1, L0_wbu_0, L0_wbu_1,
           L1_lat1_0_dw, L1_lat1_0_pw, L1_lat1_0_bn_scale, L1_lat1_0_bn_shift,
           L1_lat1_1_dw, L1_lat1_1_pw, L1_lat1_1_bn_scale, L1_lat1_1_bn_shift,
           L1_lat2_0_dw, L1_lat2_0_pw, L1_lat2_0_bn_scale, L1_lat2_0_bn_shift,
           L1_lat2_1_dw, L1_lat2_1_pw, L1_lat2_1_bn_scale, L1_lat2_1_bn_shift,
           L1_wtd_0, L1_wtd_1, L1_wbu_0, L1_wbu_1):
    n = input0.shape[0]
    f32 = jnp.float32

    xr = input0.reshape(n, 8, 128, 8, 16)
    x0 = jnp.transpose(xr, (0, 2, 3, 4, 1)).reshape(n, 32, 4, 8, 128)
    xr = input1.reshape(n, 16, 64, 8, 8)
    x1 = jnp.transpose(xr, (0, 2, 3, 4, 1)).reshape(n, 32, 2, 8, 128)
    xp = jnp.pad(input2, ((0, 0), (0, 8), (0, 0), (0, 0)))
    x2 = jnp.transpose(xp, (0, 2, 3, 1)).reshape(n, 32, 8, 128)

    eye16 = jnp.eye(16, dtype=f32)
    eye8 = jnp.eye(8, dtype=f32)
    lw0 = jnp.concatenate(
        [jnp.kron(eye16[:, 4 * r:4 * r + 4], plat0_w) for r in (0, 2, 1, 3)],
        axis=0)
    lw1 = jnp.concatenate(
        [jnp.kron(eye8[:, 4 * r:4 * r + 4], plat1_w) for r in (0, 1)],
        axis=0)
    lw2 = _kron4(jnp.pad(plat2_w, ((0, 8), (0, 0))))
    lb = jnp.stack([_tile4(plat0_b), _tile4(plat1_b), _tile4(plat2_b)])

    se = jnp.array([[1, 1, 0, 0], [0, 0, 1, 1], [0, 0, 0, 0], [0, 0, 0, 0]], f32)
    so = jnp.array([[0, 0, 0, 0], [0, 0, 0, 0], [1, 1, 0, 0], [0, 0, 1, 1]], f32)
    sa = jnp.array([[1, 0, 0, 0], [0, 0, 0, 0], [0, 1, 0, 0], [0, 0, 0, 0]], f32)
    sb = jnp.array([[0, 0, 1, 0], [0, 0, 0, 0], [0, 0, 0, 1], [0, 0, 0, 0]], f32)
    eye32 = jnp.eye(32, dtype=f32)
    rs = jnp.concatenate([jnp.kron(s, eye32) for s in (se, so, sa, sb)], axis=0)

    steps = [
        (L0_lat1_0_dw, L0_lat1_0_pw, L0_lat1_0_bn_scale, L0_lat1_0_bn_shift, L0_wtd_0),
        (L0_lat1_1_dw, L0_lat1_1_pw, L0_lat1_1_bn_scale, L0_lat1_1_bn_shift, L0_wtd_1),
        (L0_lat2_0_dw, L0_lat2_0_pw, L0_lat2_0_bn_scale, L0_lat2_0_bn_shift, L0_wbu_0),
        (L0_lat2_1_dw, L0_lat2_1_pw, L0_lat2_1_bn_scale, L0_lat2_1_bn_shift, L0_wbu_1),
        (L1_lat1_0_dw, L1_lat1_0_pw, L1_lat1_0_bn_scale, L1_lat1_0_bn_shift, L1_wtd_0),
        (L1_lat1_1_dw, L1_lat1_1_pw, L1_lat1_1_bn_scale, L1_lat1_1_bn_shift, L1_wtd_1),
        (L1_lat2_0_dw, L1_lat2_0_pw, L1_lat2_0_bn_scale, L1_lat2_0_bn_shift, L1_wbu_0),
        (L1_lat2_1_dw, L1_lat2_1_pw, L1_lat2_1_bn_scale, L1_lat2_1_bn_shift, L1_wbu_1),
    ]
    pws = jnp.concatenate([_kron4(s[1]) for s in steps], axis=0)
    dws = jnp.stack([_tile4(s[0]) for s in steps])
    bns = jnp.stack([_tile4(s[2]) for s in steps])
    bnt = jnp.stack([_tile4(s[3]) for s in steps])
    wn = jnp.stack([_wn(s[4]) for s in steps])

    const = lambda i, w_: (0, 0)
    o0, o1, o2 = pl.pallas_call(
        _bifpn_body,
        out_shape=[
            jax.ShapeDtypeStruct((n, 32, 4, 32, 128), f32),
            jax.ShapeDtypeStruct((n, 32, 2, 16, 128), f32),
            jax.ShapeDtypeStruct((n, 32, 8, 128), f32),
        ],
        grid_spec=pltpu.PrefetchScalarGridSpec(
            num_scalar_prefetch=1,
            grid=(n,),
            in_specs=[
                pl.BlockSpec((1, 32, 4, 8, 128), lambda i, w_: (i, 0, 0, 0, 0)),
                pl.BlockSpec((1, 32, 2, 8, 128), lambda i, w_: (i, 0, 0, 0, 0)),
                pl.BlockSpec((1, 32, 8, 128), lambda i, w_: (i, 0, 0, 0)),
                pl.BlockSpec((512, 128), const),
                pl.BlockSpec((256, 128), const),
                pl.BlockSpec((128, 128), const),
                pl.BlockSpec((3, 128), const),
                pl.BlockSpec((512, 128), const),
                pl.BlockSpec((1024, 128), const),
                pl.BlockSpec((8, 128), const),
                pl.BlockSpec((8, 128), const),
                pl.BlockSpec((8, 128), const),
            ],
            out_specs=[
                pl.BlockSpec((1, 32, 4, 32, 128), lambda i, w_: (i, 0, 0, 0, 0)),
                pl.BlockSpec((1, 32, 2, 16, 128), lambda i, w_: (i, 0, 0, 0, 0)),
                pl.BlockSpec((1, 32, 8, 128), lambda i, w_: (i, 0, 0, 0)),
            ],
        ),
        compiler_params=pltpu.CompilerParams(
            dimension_semantics=("parallel",),
            vmem_limit_bytes=64 * 1024 * 1024,
        ),
    )(wn, x0, x1, x2, lw0, lw1, lw2, lb, rs, pws, dws, bns, bnt)

    y0 = o0.reshape(n, 128, 2, 2, 8, 4, 32)
    y0 = jnp.transpose(y0, (0, 6, 1, 4, 3, 2, 5)).reshape(n, 32, 128, 128)
    y1 = o1.reshape(n, 64, 2, 8, 4, 32)
    y1 = jnp.transpose(y1, (0, 5, 1, 3, 2, 4)).reshape(n, 32, 64, 64)
    y2 = o2.reshape(n, 32, 8, 4, 32)
    y2 = jnp.transpose(y2, (0, 4, 1, 2, 3)).reshape(n, 32, 32, 32)
    return [y2, y1, y0]


# speedup vs baseline: 1.4786x; 1.2566x over previous
import jax
import jax.numpy as jnp
from jax.experimental import pallas as pl
from jax.experimental.pallas import tpu as pltpu

_EPS_FUSED = 1e-4


def _mm(x, w):
    h, g, k = x.shape
    y = jnp.dot(x.reshape(h * g, k), w, preferred_element_type=jnp.float32)
    return y.reshape(h, g, 128)


def _bifpn_body(wn_ref, x0_ref, x1_ref, x2_ref, lw0_ref, lw1_ref, lw2_ref,
                lb_ref, rs_ref, pws_ref, dws_ref, bns_ref, bnt_ref,
                o0_ref, o1_ref, o2_ref):
    re_m = rs_ref[0:128, :]
    ro_m = rs_ref[128:256, :]
    da_m = rs_ref[256:384, :]
    db_m = rs_ref[384:512, :]

    def up2(x):
        xx = jnp.concatenate([x, x], axis=0)
        return jnp.concatenate([_mm(xx, re_m), _mm(xx, ro_m)], axis=1)

    def down2(x):
        h, g, _ = x.shape
        a = x[:h // 2, :g // 2]
        b = x[:h // 2, g // 2:]
        return _mm(a, da_m) + _mm(b, db_m)

    def fused(plist, k):
        acc = wn_ref[k, 0] * plist[0]
        for j in range(1, len(plist)):
            acc = acc + wn_ref[k, j] * plist[j]
        x = acc * dws_ref[k:k + 1, :]
        y = _mm(x, pws_ref[128 * k:128 * (k + 1), :])
        y = y * bns_ref[k:k + 1, :] + bnt_ref[k:k + 1, :]
        return jnp.maximum(y, 0.0)

    x0r = x0_ref[0]
    x0s = jnp.concatenate(
        [x0r[:, 0], x0r[:, 2], x0r[:, 1], x0r[:, 3]], axis=0)
    x1r = x1_ref[0]
    x1s = jnp.concatenate([x1r[:, 0], x1r[:, 1]], axis=0)

    p0 = jnp.concatenate(
        [_mm(x0s, lw0_ref[0:128, :]), _mm(x0s, lw0_ref[128:256, :]),
         _mm(x0s, lw0_ref[256:384, :]), _mm(x0s, lw0_ref[384:512, :])],
        axis=1) + lb_ref[0:1, :]
    p1 = jnp.concatenate(
        [_mm(x1s, lw1_ref[0:128, :]), _mm(x1s, lw1_ref[128:256, :])],
        axis=1) + lb_ref[1:2, :]
    p2 = _mm(x2_ref[0], lw2_ref[...]) + lb_ref[2:3, :]

    ps = [p2, p1, p0]
    for l in range(2):
        base = 4 * l
        a2 = ps[0]
        a1 = fused([ps[1], up2(a2)], base + 0)
        a0 = fused([ps[2], up2(a1)], base + 1)
        o1 = fused([ps[1], a1, down2(a0)], base + 2)
        o2 = fused([ps[0], a2, down2(o1)], base + 3)
        ps = [o2, o1, a0]

    s0, s1 = ps[2], ps[1]
    perm4 = (0, 2, 1, 3)
    for rh in range(4):
        for rg in range(4):
            o0_ref[0, :, rh, :, rg] = s0[32 * perm4[rh]:32 * perm4[rh] + 32,
                                         8 * perm4[rg]:8 * perm4[rg] + 8]
    for rh in range(2):
        for rg in range(2):
            o1_ref[0, :, rh, :, rg] = s1[32 * rh:32 * rh + 32,
                                         8 * rg:8 * rg + 8]
    o2_ref[0] = ps[0]


def _kron4(m):
    return jnp.kron(jnp.eye(4, dtype=m.dtype), m)


def _tile4(v):
    return jnp.tile(v, 4)


def _wn(w_raw):
    w = jnp.maximum(w_raw, 0.0)
    w = w / (jnp.sum(w) + _EPS_FUSED)
    return jnp.pad(w, (0, 3 - w.shape[0]))


def kernel(input0, input1, input2, plat0_w, plat0_b, plat1_w, plat1_b,
           plat2_w, plat2_b, L0_lat1_0_dw, L0_lat1_0_pw, L0_lat1_0_bn_scale,
           L0_lat1_0_bn_shift, L0_lat1_1_dw, L0_lat1_1_pw, L0_lat1_1_bn_scale,
           L0_lat1_1_bn_shift, L0_lat2_0_dw, L0_lat2_0_pw, L0_lat2_0_bn_scale,
           L0_lat2_0_bn_shift, L0_lat2_1_dw, L0_lat2_1_pw, L0_lat2_1_bn_scale,
           L0_lat2_1_bn_shift, L0_wtd_0, L0_wtd_1, L0_wbu_0, L0_wbu_1,
           L1_lat1_0_dw, L1_lat1_0_pw, L1_lat1_0_bn_scale, L1_lat1_0_bn_shift,
           L1_lat1_1_dw, L1_lat1_1_pw, L1_lat1_1_bn_scale, L1_lat1_1_bn_shift,
           L1_lat2_0_dw, L1_lat2_0_pw, L1_lat2_0_bn_scale, L1_lat2_0_bn_shift,
           L1_lat2_1_dw, L1_lat2_1_pw, L1_lat2_1_bn_scale, L1_lat2_1_bn_shift,
           L1_wtd_0, L1_wtd_1, L1_wbu_0, L1_wbu_1):
    n = input0.shape[0]
    f32 = jnp.float32

    xr = input0.reshape(n, 8, 128, 8, 16)
    x0 = jnp.transpose(xr, (0, 2, 3, 4, 1)).reshape(n, 32, 4, 8, 128)
    xr = input1.reshape(n, 16, 64, 8, 8)
    x1 = jnp.transpose(xr, (0, 2, 3, 4, 1)).reshape(n, 32, 2, 8, 128)
    xp = jnp.pad(input2, ((0, 0), (0, 8), (0, 0), (0, 0)))
    x2 = jnp.transpose(xp, (0, 2, 3, 1)).reshape(n, 32, 8, 128)

    eye16 = jnp.eye(16, dtype=f32)
    eye8 = jnp.eye(8, dtype=f32)
    lw0 = jnp.concatenate(
        [jnp.kron(eye16[:, 4 * r:4 * r + 4], plat0_w) for r in (0, 2, 1, 3)],
        axis=0)
    lw1 = jnp.concatenate(
        [jnp.kron(eye8[:, 4 * r:4 * r + 4], plat1_w) for r in (0, 1)],
        axis=0)
    lw2 = _kron4(jnp.pad(plat2_w, ((0, 8), (0, 0))))
    lb = jnp.stack([_tile4(plat0_b), _tile4(plat1_b), _tile4(plat2_b)])

    se = jnp.array([[1, 1, 0, 0], [0, 0, 1, 1], [0, 0, 0, 0], [0, 0, 0, 0]], f32)
    so = jnp.array([[0, 0, 0, 0], [0, 0, 0, 0], [1, 1, 0, 0], [0, 0, 1, 1]], f32)
    sa = jnp.array([[1, 0, 0, 0], [0, 0, 0, 0], [0, 1, 0, 0], [0, 0, 0, 0]], f32)
    sb = jnp.array([[0, 0, 1, 0], [0, 0, 0, 0], [0, 0, 0, 1], [0, 0, 0, 0]], f32)
    eye32 = jnp.eye(32, dtype=f32)
    rs = jnp.concatenate([jnp.kron(s, eye32) for s in (se, so, sa, sb)], axis=0)

    steps = [
        (L0_lat1_0_dw, L0_lat1_0_pw, L0_lat1_0_bn_scale, L0_lat1_0_bn_shift, L0_wtd_0),
        (L0_lat1_1_dw, L0_lat1_1_pw, L0_lat1_1_bn_scale, L0_lat1_1_bn_shift, L0_wtd_1),
        (L0_lat2_0_dw, L0_lat2_0_pw, L0_lat2_0_bn_scale, L0_lat2_0_bn_shift, L0_wbu_0),
        (L0_lat2_1_dw, L0_lat2_1_pw, L0_lat2_1_bn_scale, L0_lat2_1_bn_shift, L0_wbu_1),
        (L1_lat1_0_dw, L1_lat1_0_pw, L1_lat1_0_bn_scale, L1_lat1_0_bn_shift, L1_wtd_0),
        (L1_lat1_1_dw, L1_lat1_1_pw, L1_lat1_1_bn_scale, L1_lat1_1_bn_shift, L1_wtd_1),
        (L1_lat2_0_dw, L1_lat2_0_pw, L1_lat2_0_bn_scale, L1_lat2_0_bn_shift, L1_wbu_0),
        (L1_lat2_1_dw, L1_lat2_1_pw, L1_lat2_1_bn_scale, L1_lat2_1_bn_shift, L1_wbu_1),
    ]
    pws = jnp.concatenate([_kron4(s[1]) for s in steps], axis=0)
    dws = jnp.stack([_tile4(s[0]) for s in steps])
    bns = jnp.stack([_tile4(s[2]) for s in steps])
    bnt = jnp.stack([_tile4(s[3]) for s in steps])
    wn = jnp.stack([_wn(s[4]) for s in steps])

    const = lambda i, w_: (0, 0)
    o0, o1, o2 = pl.pallas_call(
        _bifpn_body,
        out_shape=[
            jax.ShapeDtypeStruct((n, 32, 4, 8, 4, 128), f32),
            jax.ShapeDtypeStruct((n, 32, 2, 8, 2, 128), f32),
            jax.ShapeDtypeStruct((n, 32, 8, 128), f32),
        ],
        grid_spec=pltpu.PrefetchScalarGridSpec(
            num_scalar_prefetch=1,
            grid=(n,),
            in_specs=[
                pl.BlockSpec((1, 32, 4, 8, 128), lambda i, w_: (i, 0, 0, 0, 0)),
                pl.BlockSpec((1, 32, 2, 8, 128), lambda i, w_: (i, 0, 0, 0, 0)),
                pl.BlockSpec((1, 32, 8, 128), lambda i, w_: (i, 0, 0, 0)),
                pl.BlockSpec((512, 128), const),
                pl.BlockSpec((256, 128), const),
                pl.BlockSpec((128, 128), const),
                pl.BlockSpec((3, 128), const),
                pl.BlockSpec((512, 128), const),
                pl.BlockSpec((1024, 128), const),
                pl.BlockSpec((8, 128), const),
                pl.BlockSpec((8, 128), const),
                pl.BlockSpec((8, 128), const),
            ],
            out_specs=[
                pl.BlockSpec((1, 32, 4, 8, 4, 128),
                             lambda i, w_: (i, 0, 0, 0, 0, 0)),
                pl.BlockSpec((1, 32, 2, 8, 2, 128),
                             lambda i, w_: (i, 0, 0, 0, 0, 0)),
                pl.BlockSpec((1, 32, 8, 128), lambda i, w_: (i, 0, 0, 0)),
            ],
        ),
        compiler_params=pltpu.CompilerParams(
            dimension_semantics=("parallel",),
            vmem_limit_bytes=64 * 1024 * 1024,
        ),
    )(wn, x0, x1, x2, lw0, lw1, lw2, lb, rs, pws, dws, bns, bnt)

    def unprep(o, h, w):
        return jnp.transpose(o.reshape(n, h, w, 32), (0, 3, 1, 2))

    return [unprep(o2, 32, 32), unprep(o1, 64, 64), unprep(o0, 128, 128)]


# speedup vs baseline: 1.7301x; 1.1702x over previous
import jax
import jax.numpy as jnp
from jax.experimental import pallas as pl
from jax.experimental.pallas import tpu as pltpu

_EPS_FUSED = 1e-4


def _mm(x, w):
    h, g, k = x.shape
    y = jnp.dot(x.reshape(h * g, k), w, preferred_element_type=jnp.float32)
    return y.reshape(h, g, 128)


def _up2(x):
    h, g, _ = x.shape
    xh = jnp.broadcast_to(x[:, None], (h, 2, g, 128)).reshape(2 * h, g, 128)
    e = jnp.concatenate(
        [xh[..., 0:32], xh[..., 0:32], xh[..., 32:64], xh[..., 32:64]], axis=-1)
    o = jnp.concatenate(
        [xh[..., 64:96], xh[..., 64:96], xh[..., 96:128], xh[..., 96:128]],
        axis=-1)
    return jnp.stack([e, o], axis=2).reshape(2 * h, 2 * g, 128)


def _down2(x):
    h, g, _ = x.shape
    xh = x.reshape(h // 2, 2, g, 128)[:, 0]
    ab = xh.reshape(h // 2, g // 2, 2, 128)
    a = ab[:, :, 0]
    b = ab[:, :, 1]
    return jnp.concatenate(
        [a[..., 0:32], a[..., 64:96], b[..., 0:32], b[..., 64:96]], axis=-1)


def _bifpn_body(wn_ref, x0_ref, x1_ref, x2_ref, lw0_ref, lw1_ref, lw2_ref,
                lb_ref, pws_ref, dws_ref, bns_ref, bnt_ref,
                o0_ref, o1_ref, o2_ref):
    def fused(plist, k):
        acc = wn_ref[k, 0] * plist[0]
        for j in range(1, len(plist)):
            acc = acc + wn_ref[k, j] * plist[j]
        x = acc * dws_ref[k:k + 1, :]
        y = _mm(x, pws_ref[128 * k:128 * (k + 1), :])
        y = y * bns_ref[k:k + 1, :] + bnt_ref[k:k + 1, :]
        return jnp.maximum(y, 0.0)

    p0 = _mm(x0_ref[0], lw0_ref[...]) + lb_ref[0:1, :]
    p1 = _mm(x1_ref[0], lw1_ref[...]) + lb_ref[1:2, :]
    p2 = _mm(x2_ref[0], lw2_ref[...]) + lb_ref[2:3, :]

    ps = [p2, p1, p0]
    for l in range(2):
        base = 4 * l
        a2 = ps[0]
        a1 = fused([ps[1], _up2(a2)], base + 0)
        a0 = fused([ps[2], _up2(a1)], base + 1)
        o1 = fused([ps[1], a1, _down2(a0)], base + 2)
        o2 = fused([ps[0], a2, _down2(o1)], base + 3)
        ps = [o2, o1, a0]

    o2_ref[0] = ps[0]
    o1_ref[0] = ps[1]
    o0_ref[0] = ps[2]


def _kron4(m):
    return jnp.kron(jnp.eye(4, dtype=m.dtype), m)


def _tile4(v):
    return jnp.tile(v, 4)


def _wn(w_raw):
    w = jnp.maximum(w_raw, 0.0)
    w = w / (jnp.sum(w) + _EPS_FUSED)
    return jnp.pad(w, (0, 3 - w.shape[0]))


def kernel(input0, input1, input2, plat0_w, plat0_b, plat1_w, plat1_b,
           plat2_w, plat2_b, L0_lat1_0_dw, L0_lat1_0_pw, L0_lat1_0_bn_scale,
           L0_lat1_0_bn_shift, L0_lat1_1_dw, L0_lat1_1_pw, L0_lat1_1_bn_scale,
           L0_lat1_1_bn_shift, L0_lat2_0_dw, L0_lat2_0_pw, L0_lat2_0_bn_scale,
           L0_lat2_0_bn_shift, L0_lat2_1_dw, L0_lat2_1_pw, L0_lat2_1_bn_scale,
           L0_lat2_1_bn_shift, L0_wtd_0, L0_wtd_1, L0_wbu_0, L0_wbu_1,
           L1_lat1_0_dw, L1_lat1_0_pw, L1_lat1_0_bn_scale, L1_lat1_0_bn_shift,
           L1_lat1_1_dw, L1_lat1_1_pw, L1_lat1_1_bn_scale, L1_lat1_1_bn_shift,
           L1_lat2_0_dw, L1_lat2_0_pw, L1_lat2_0_bn_scale, L1_lat2_0_bn_shift,
           L1_lat2_1_dw, L1_lat2_1_pw, L1_lat2_1_bn_scale, L1_lat2_1_bn_shift,
           L1_wtd_0, L1_wtd_1, L1_wbu_0, L1_wbu_1):
    n = input0.shape[0]

    def prep_x(x):
        _, c, h, w = x.shape
        return jnp.transpose(x, (0, 2, 3, 1)).reshape(n, h, w // 4, 4 * c)

    x0 = prep_x(input0)
    x1 = prep_x(input1)
    x2 = prep_x(input2)

    lw0 = _kron4(plat0_w)
    lw1 = _kron4(plat1_w)
    lw2 = _kron4(plat2_w)
    lb = jnp.stack([_tile4(plat0_b), _tile4(plat1_b), _tile4(plat2_b)])

    steps = [
        (L0_lat1_0_dw, L0_lat1_0_pw, L0_lat1_0_bn_scale, L0_lat1_0_bn_shift, L0_wtd_0),
        (L0_lat1_1_dw, L0_lat1_1_pw, L0_lat1_1_bn_scale, L0_lat1_1_bn_shift, L0_wtd_1),
        (L0_lat2_0_dw, L0_lat2_0_pw, L0_lat2_0_bn_scale, L0_lat2_0_bn_shift, L0_wbu_0),
        (L0_lat2_1_dw, L0_lat2_1_pw, L0_lat2_1_bn_scale, L0_lat2_1_bn_shift, L0_wbu_1),
        (L1_lat1_0_dw, L1_lat1_0_pw, L1_lat1_0_bn_scale, L1_lat1_0_bn_shift, L1_wtd_0),
        (L1_lat1_1_dw, L1_lat1_1_pw, L1_lat1_1_bn_scale, L1_lat1_1_bn_shift, L1_wtd_1),
        (L1_lat2_0_dw, L1_lat2_0_pw, L1_lat2_0_bn_scale, L1_lat2_0_bn_shift, L1_wbu_0),
        (L1_lat2_1_dw, L1_lat2_1_pw, L1_lat2_1_bn_scale, L1_lat2_1_bn_shift, L1_wbu_1),
    ]
    pws = jnp.concatenate([_kron4(s[1]) for s in steps], axis=0)
    dws = jnp.stack([_tile4(s[0]) for s in steps])
    bns = jnp.stack([_tile4(s[2]) for s in steps])
    bnt = jnp.stack([_tile4(s[3]) for s in steps])
    wn = jnp.stack([_wn(s[4]) for s in steps])

    const = lambda i, w_: (0, 0)
    o0, o1, o2 = pl.pallas_call(
        _bifpn_body,
        out_shape=[
            jax.ShapeDtypeStruct((n, 128, 32, 128), jnp.float32),
            jax.ShapeDtypeStruct((n, 64, 16, 128), jnp.float32),
            jax.ShapeDtypeStruct((n, 32, 8, 128), jnp.float32),
        ],
        grid_spec=pltpu.PrefetchScalarGridSpec(
            num_scalar_prefetch=1,
            grid=(n,),
            in_specs=[
                pl.BlockSpec((1, 128, 32, 32), lambda i, w_: (i, 0, 0, 0)),
                pl.BlockSpec((1, 64, 16, 64), lambda i, w_: (i, 0, 0, 0)),
                pl.BlockSpec((1, 32, 8, 96), lambda i, w_: (i, 0, 0, 0)),
                pl.BlockSpec((32, 128), const),
                pl.BlockSpec((64, 128), const),
                pl.BlockSpec((96, 128), const),
                pl.BlockSpec((3, 128), const),
                pl.BlockSpec((1024, 128), const),
                pl.BlockSpec((8, 128), const),
                pl.BlockSpec((8, 128), const),
                pl.BlockSpec((8, 128), const),
            ],
            out_specs=[
                pl.BlockSpec((1, 128, 32, 128), lambda i, w_: (i, 0, 0, 0)),
                pl.BlockSpec((1, 64, 16, 128), lambda i, w_: (i, 0, 0, 0)),
                pl.BlockSpec((1, 32, 8, 128), lambda i, w_: (i, 0, 0, 0)),
            ],
        ),
        compiler_params=pltpu.CompilerParams(
            dimension_semantics=("parallel",),
            vmem_limit_bytes=64 * 1024 * 1024,
        ),
    )(wn, x0, x1, x2, lw0, lw1, lw2, lb, pws, dws, bns, bnt)

    def unprep(o, h, w):
        return jnp.transpose(o.reshape(n, h, w, 32), (0, 3, 1, 2))

    return [unprep(o2, 32, 32), unprep(o1, 64, 64), unprep(o0, 128, 128)]
